# Initial kernel scaffold; baseline (speedup 1.0000x reference)
import jax
import jax.numpy as jnp
from jax.experimental import pallas as pl


def kernel(x, edge_index, edge_attr, batch, W_l, b_l, W_r, b_r, W_e, att, conv_bias, pool_p, W1, b1, W2, b2, W3, b3):
    raise NotImplementedError("write your pallas kernel here")


# speedup vs baseline: 1.0507x; 1.0507x over previous
import functools

import jax
import jax.numpy as jnp
from jax.experimental import pallas as pl
from jax.experimental.pallas import tpu as pltpu

N = 10000
E = 160000
D_IN = 128
D_EDGE = 16
H = 8
C = 64
HC = H * C
G = 16
RATIO = 0.8
OUT = 10
NEG = 0.2


def _matmul_kern(x_ref, w_ref, b_ref, o_ref):
    o_ref[...] = (
        jnp.dot(x_ref[...], w_ref[...], preferred_element_type=jnp.float32)
        + b_ref[...]
    )


def _proj(x, W, b, block_rows=1024):
    n = x.shape[0]
    n_pad = ((n + block_rows - 1) // block_rows) * block_rows
    if n_pad != n:
        x = jnp.pad(x, ((0, n_pad - n), (0, 0)))
    out = pl.pallas_call(
        _matmul_kern,
        grid=(n_pad // block_rows,),
        in_specs=[
            pl.BlockSpec((block_rows, x.shape[1]), lambda i: (i, 0)),
            pl.BlockSpec((x.shape[1], W.shape[1]), lambda i: (0, 0)),
            pl.BlockSpec((1, W.shape[1]), lambda i: (0, 0)),
        ],
        out_specs=pl.BlockSpec((block_rows, W.shape[1]), lambda i: (i, 0)),
        out_shape=jax.ShapeDtypeStruct((n_pad, W.shape[1]), jnp.float32),
    )(x, W, b[None, :])
    return out[:n]


def kernel(x, edge_index, edge_attr, batch, W_l, b_l, W_r, b_r, W_e, att,
           conv_bias, pool_p, W1, b1, W2, b2, W3, b3):
    n = x.shape[0]
    loop = jnp.arange(n, dtype=edge_index.dtype)
    src = jnp.concatenate([edge_index[0], loop])
    dst = jnp.concatenate([edge_index[1], loop])
    ea_mean = jnp.mean(edge_attr, axis=0, keepdims=True)

    x_l = _proj(x, W_l, b_l).reshape(n, H, C)
    x_r = _proj(x, W_r, b_r).reshape(n, H, C)
    e_real = _proj(edge_attr, W_e, jnp.zeros((HC,), jnp.float32))
    e_self = (ea_mean @ W_e)
    e_f = jnp.concatenate(
        [e_real, jnp.tile(e_self, (n, 1))], axis=0).reshape(-1, H, C)

    m = x_l[src] + x_r[dst] + e_f
    m = jax.nn.leaky_relu(m, negative_slope=NEG)
    logits = jnp.sum(m * att[None, :, :], axis=-1)
    ex = jnp.exp(logits)
    denom = jax.ops.segment_sum(ex, dst, num_segments=n)
    num = jax.ops.segment_sum(x_l[src] * ex[:, :, None], dst, num_segments=n)
    out = num / (denom + 1e-16)[:, :, None]
    h = jax.nn.relu(out.reshape(n, HC) + conv_bias)

    score = jnp.tanh((h * pool_p).sum(axis=-1) / jnp.linalg.norm(pool_p))
    counts = jnp.bincount(batch, length=G)
    k = jnp.ceil(RATIO * counts).astype(jnp.int32)
    order = jnp.lexsort((-score, batch))
    starts = jnp.concatenate([jnp.zeros((1,), counts.dtype), jnp.cumsum(counts)[:-1]])
    rank = jnp.arange(n) - starts[batch[order]]
    keep_sorted = rank < k[batch[order]]
    mask = jnp.zeros((n,), bool).at[order].set(keep_sorted)

    val = h * score[:, None]
    gmp = jax.ops.segment_max(jnp.where(mask[:, None], val, jnp.float32(-1e30)), batch, num_segments=G)
    msk = mask.astype(jnp.float32)
    gap = jax.ops.segment_sum(val * msk[:, None], batch, num_segments=G) / (
        jax.ops.segment_sum(msk, batch, num_segments=G)[:, None] + 1e-16)
    cont = jnp.concatenate([gmp, gap], axis=1)

    z = jax.nn.relu(cont @ W1 + b1)
    z = jax.nn.relu(z @ W2 + b2)
    return jax.nn.log_softmax(z @ W3 + b3, axis=-1)


# speedup vs baseline: 7.4046x; 7.0473x over previous
import jax
import jax.numpy as jnp
from jax import lax
from jax.experimental import pallas as pl
from jax.experimental.pallas import tpu as pltpu
from jax.experimental.pallas import tpu_sc as plsc

N = 10000
NP = 10240
E = 160000
D_IN = 128
D_EDGE = 16
H = 8
C = 64
HC = H * C
G = 16
RATIO = 0.8
OUT = 10
NEG = 0.2

CHUNK = 512
NCHUNK = 21
EP = 16 * NCHUNK * CHUNK
EFR = 160768
XR0 = 10240
EF0 = 2 * 10240
TR = EF0 + EFR
RANGE = 1024
NROUND = NP // (2 * RANGE)
TROWS = RANGE // 16
FB = 16
B = 16
SELCAP = NCHUNK * CHUNK + 4 * B


def _perm_cols(W):
    return W.reshape(-1, H, C).transpose(0, 2, 1).reshape(-1, HC)


def _perm_vec(v):
    return v.reshape(H, C).T.reshape(HC)


def _perm_rows(M):
    return M.reshape(H, C, -1).transpose(1, 0, 2).reshape(HC, -1)


def _bcast_swap8(v):
    idx = jax.lax.iota(jnp.int32, 16) ^ 8
    dnums = lax.GatherDimensionNumbers(
        offset_dims=(), collapsed_slice_dims=(0,), start_index_map=(0,))
    return lax.gather(v, idx[:, None], dnums, (1,),
                      mode=lax.GatherScatterMode.PROMISE_IN_BOUNDS)


def _edge_body(tab, src3, attA, biasA, h_out,
               idx_c, sel_src, sel_dstl, sel_efi,
               idx48, dstl16,
               rows48, numb0, numb1, numb2, numb3, numb4,
               attv, biasv,
               sh0, sh1, sh2, sh3, sh4, sem0, sem1, sem2):
    c = lax.axis_index("c")
    s = lax.axis_index("s")
    numb = (numb0, numb1, numb2, numb3, numb4)
    sh = (sh0, sh1, sh2, sh3, sh4)
    pltpu.sync_copy(attA, attv)
    pltpu.sync_copy(biasA, biasv)
    zf16 = jnp.zeros((16,), jnp.float32)
    zi16 = jnp.zeros((16,), jnp.int32)

    def _round(r, _0):
        lo = (2 * r + c) * RANGE

        def _zrow(i, _):
            for g in range(5):
                for j in range(8):
                    numb[g][i, pl.ds(j * 16, 16)] = zf16
            return 0
        lax.fori_loop(0, FB, _zrow, 0)

        def _zcopy(f, _):
            cps = [pltpu.async_copy(
                numb[g], sh[g].at[pl.ds(s * TROWS + f * FB, FB)], sem1)
                for g in range(5)]
            for cp in cps:
                cp.wait()
            return 0
        lax.fori_loop(0, TROWS // FB, _zcopy, 0)
        plsc.subcore_barrier()

        def _chunk(ch, cnt):
            base = (s * NCHUNK + ch) * (3 * CHUNK)
            pltpu.sync_copy(src3.at[pl.ds(base, 3 * CHUNK)], idx_c)
            for v in range(CHUNK // 16):
                d = idx_c[pl.ds(CHUNK + v * 16, 16)]
                m = (d >= lo) & (d < lo + RANGE)
                plsc.store_compressed(sel_src.at[pl.ds(cnt, 16)],
                                      idx_c[pl.ds(v * 16, 16)], mask=m)
                plsc.store_compressed(sel_dstl.at[pl.ds(cnt, 16)], d - lo,
                                      mask=m)
                plsc.store_compressed(sel_efi.at[pl.ds(cnt, 16)],
                                      idx_c[pl.ds(2 * CHUNK + v * 16, 16)],
                                      mask=m)
                cnt = cnt + jnp.sum(m.astype(jnp.int32))
            return cnt
        cnt = lax.fori_loop(0, NCHUNK, _chunk, jnp.int32(0))
        for t in (0, 16, 32, 48):
            sel_src[pl.ds(cnt + t, 16)] = zi16
            sel_dstl[pl.ds(cnt + t, 16)] = zi16
            sel_efi[pl.ds(cnt + t, 16)] = zi16

        nsub = (cnt + B - 1) // B

        def _sub(sb, _):
            b0 = sb * B
            dstl = sel_dstl[pl.ds(b0, 16)]
            dstl16[pl.ds(0, 16)] = dstl
            idx48[pl.ds(0, 16)] = sel_src[pl.ds(b0, 16)]
            idx48[pl.ds(16, 16)] = dstl + (lo + XR0)
            idx48[pl.ds(32, 16)] = sel_efi[pl.ds(b0, 16)] + EF0
            pltpu.async_copy(tab.at[idx48], rows48, sem0).wait()

            def _edge(i, _):
                acc = zf16
                for j in range(HC // 16):
                    jsl = pl.ds(j * 16, 16)
                    z = (rows48[i, jsl] + rows48[i + 16, jsl]
                         + rows48[i + 32, jsl])
                    z = jnp.where(z >= 0.0, z, z * NEG)
                    acc = acc + z * attv[jsl]
                hs = acc + _bcast_swap8(acc)
                valid = ((b0 + i) < cnt).astype(jnp.float32)
                ex = jnp.exp(hs) * valid
                numb4[i, pl.ds(0, 16)] = ex
                for j in range(HC // 16):
                    jsl = pl.ds(j * 16, 16)
                    numb[j // 8][i, pl.ds((j % 8) * 16, 16)] = (
                        rows48[i, jsl] * ex)
                return 0
            lax.fori_loop(0, B, _edge, 0)
            cps = [pltpu.async_copy(numb[g], sh[g].at[dstl16], sem2,
                                    add=True)
                   for g in range(5)]
            for cp in cps:
                cp.wait()
            return 0
        lax.fori_loop(0, nsub, _sub, 0)
        plsc.subcore_barrier()

        def _flush(f, _):
            r0 = s * TROWS + f * FB
            cps = [pltpu.async_copy(sh[g].at[pl.ds(r0, FB)], numb[g], sem1)
                   for g in range(5)]
            for cp in cps:
                cp.wait()

            def _row(row, _):
                rec = 1.0 / (numb4[row, pl.ds(0, 16)] + 1e-30)
                for j in range(HC // 16):
                    jsl = pl.ds(j * 16, 16)
                    rows48[row, jsl] = jnp.maximum(
                        numb[j // 8][row, pl.ds((j % 8) * 16, 16)] * rec
                        + biasv[jsl], 0.0)
                return 0
            lax.fori_loop(0, FB, _row, 0)
            pltpu.sync_copy(rows48.at[pl.ds(0, FB)],
                            h_out.at[pl.ds(lo + r0, FB)])
            return 0
        lax.fori_loop(0, TROWS // FB, _flush, 0)
        plsc.subcore_barrier()
        return 0

    lax.fori_loop(0, NROUND, _round, 0)


def _gat_conv_sc(tab, src3, att_p, bias_p):
    mesh = plsc.VectorSubcoreMesh(core_axis_name="c", subcore_axis_name="s")
    f32 = jnp.float32
    i32 = jnp.int32
    return pl.kernel(
        _edge_body,
        out_type=jax.ShapeDtypeStruct((NP, HC), f32),
        mesh=mesh,
        compiler_params=pltpu.CompilerParams(needs_layout_passes=False),
        scratch_types=[
            pltpu.VMEM((3 * CHUNK,), i32),
            pltpu.VMEM((SELCAP,), i32),
            pltpu.VMEM((SELCAP,), i32),
            pltpu.VMEM((SELCAP,), i32),
            pltpu.VMEM((3 * B,), i32),
            pltpu.VMEM((16,), i32),
            pltpu.VMEM((3 * B, HC), f32),
            pltpu.VMEM((FB, 128), f32),
            pltpu.VMEM((FB, 128), f32),
            pltpu.VMEM((FB, 128), f32),
            pltpu.VMEM((FB, 128), f32),
            pltpu.VMEM((FB, 128), f32),
            pltpu.VMEM((HC,), f32),
            pltpu.VMEM((HC,), f32),
            pltpu.VMEM_SHARED((RANGE, 128), f32),
            pltpu.VMEM_SHARED((RANGE, 128), f32),
            pltpu.VMEM_SHARED((RANGE, 128), f32),
            pltpu.VMEM_SHARED((RANGE, 128), f32),
            pltpu.VMEM_SHARED((RANGE, 128), f32),
            pltpu.SemaphoreType.DMA,
            pltpu.SemaphoreType.DMA,
            pltpu.SemaphoreType.DMA,
        ],
    )(tab, src3, att_p, bias_p)


def _fused_proj_kern(x_ref, ea_ref, wl_ref, wr_ref, we_ref, bl_ref, br_ref,
                     o_ref):
    i = pl.program_id(0)

    @pl.when(i < 10)
    def _xl():
        o_ref[...] = jnp.dot(x_ref[...], wl_ref[...],
                             preferred_element_type=jnp.float32) + bl_ref[...]

    @pl.when((i >= 10) & (i < 20))
    def _xr():
        o_ref[...] = jnp.dot(x_ref[...], wr_ref[...],
                             preferred_element_type=jnp.float32) + br_ref[...]

    @pl.when(i >= 20)
    def _ef():
        o_ref[...] = jnp.dot(ea_ref[...], we_ref[...],
                             preferred_element_type=jnp.float32)


def _fused_proj(x_pad, ea_pad, W_lp, W_rp, W_ep, b_lp, b_rp):
    nb = TR // 1024
    return pl.pallas_call(
        _fused_proj_kern,
        grid=(nb,),
        in_specs=[
            pl.BlockSpec((1024, D_IN),
                         lambda i: (jnp.where(i < 10, i,
                                              jnp.where(i < 20, i - 10, 0)),
                                    0)),
            pl.BlockSpec((1024, D_EDGE),
                         lambda i: (jnp.where(i >= 20, i - 20, 0), 0)),
            pl.BlockSpec((D_IN, HC), lambda i: (0, 0)),
            pl.BlockSpec((D_IN, HC), lambda i: (0, 0)),
            pl.BlockSpec((D_EDGE, HC), lambda i: (0, 0)),
            pl.BlockSpec((1, HC), lambda i: (0, 0)),
            pl.BlockSpec((1, HC), lambda i: (0, 0)),
        ],
        out_specs=pl.BlockSpec((1024, HC), lambda i: (i, 0)),
        out_shape=jax.ShapeDtypeStruct((TR, HC), jnp.float32),
    )(x_pad, ea_pad, W_lp, W_rp, W_ep, b_lp[None, :], b_rp[None, :])


def kernel(x, edge_index, edge_attr, batch, W_l, b_l, W_r, b_r, W_e, att,
           conv_bias, pool_p, W1, b1, W2, b2, W3, b3):
    n = N
    loop = jnp.arange(n, dtype=edge_index.dtype)
    pad = EP - (E + n)
    src_all = jnp.concatenate(
        [edge_index[0], loop, jnp.zeros((pad,), jnp.int32)])
    dst_s = jnp.concatenate(
        [edge_index[1], loop, jnp.full((pad,), 1 << 30, jnp.int32)])
    efi = jnp.concatenate(
        [jnp.arange(E, dtype=jnp.int32), jnp.full((n,), E, jnp.int32),
         jnp.zeros((pad,), jnp.int32)])
    src3 = (jnp.stack([src_all, dst_s, efi], axis=0)
            .reshape(3, EP // CHUNK, CHUNK).transpose(1, 0, 2).reshape(-1))

    W_lp, b_lp = _perm_cols(W_l), _perm_vec(b_l)
    W_rp, b_rp = _perm_cols(W_r), _perm_vec(b_r)
    W_ep = _perm_cols(W_e)
    att_p = att.T.reshape(HC)
    bias_p = _perm_vec(conv_bias)
    pool_pp = _perm_vec(pool_p)

    x_pad = jnp.pad(x, ((0, NP - N), (0, 0)))
    ea_mean = jnp.mean(edge_attr, axis=0, keepdims=True)
    ea_pad = jnp.concatenate(
        [edge_attr, jnp.tile(ea_mean, (EFR - E, 1))], axis=0)
    tab = _fused_proj(x_pad, ea_pad, W_lp, W_rp, W_ep, b_lp, b_rp)

    h_full = _gat_conv_sc(tab, src3, att_p, bias_p)

    score2d = _score_tc(h_full, pool_pp)
    batch_p = jnp.concatenate(
        [batch, jnp.full((NP - N,), 1 << 20, jnp.int32)]).reshape(NP, 1)
    keep2d = _rank_tc(score2d, batch_p)
    W1_p = jnp.concatenate([_perm_rows(W1[:HC]), _perm_rows(W1[HC:])], axis=0)
    W3p = jnp.pad(W3, ((0, 0), (0, 128 - OUT)))
    b3p = jnp.pad(b3, (0, 128 - OUT), constant_values=-1e30)
    out = _pool_mlp_tc(h_full, score2d, keep2d, batch_p,
                       W1_p, b1.reshape(1, -1), W2, b2.reshape(1, -1),
                       W3p, b3p.reshape(1, -1))
    return out[:, :OUT]


BI = 256
BJ = 2048
NI = NP // BI
NJ = NP // BJ


def _score_kern(h_ref, p_ref, o_ref):
    p = p_ref[...]
    norm = jnp.sqrt(jnp.sum(p * p))
    o_ref[...] = jnp.tanh(
        jnp.dot(h_ref[...], p, preferred_element_type=jnp.float32) / norm)


def _score_tc(h_full, pool_pp):
    return pl.pallas_call(
        _score_kern,
        grid=(NI,),
        in_specs=[
            pl.BlockSpec((BI, HC), lambda i: (i, 0)),
            pl.BlockSpec((HC, 1), lambda i: (0, 0)),
        ],
        out_specs=pl.BlockSpec((BI, 1), lambda i: (i, 0)),
        out_shape=jax.ShapeDtypeStruct((NP, 1), jnp.float32),
    )(h_full, pool_pp.reshape(HC, 1))


def _rank_kern(si_ref, sj_ref, bi_ref, bj_ref, keep_ref, rank_acc, cnt_acc):
    i = pl.program_id(0)
    j = pl.program_id(1)

    @pl.when(j == 0)
    def _init():
        rank_acc[...] = jnp.zeros_like(rank_acc)
        cnt_acc[...] = jnp.zeros_like(cnt_acc)

    si = si_ref[...]
    sj = sj_ref[...]
    bi = bi_ref[...]
    bj = bj_ref[...]
    ii = (jax.lax.broadcasted_iota(jnp.int32, (BI, 1), 0)
          + i * BI).astype(jnp.float32)
    jj = (jax.lax.broadcasted_iota(jnp.int32, (1, BJ), 1)
          + j * BJ).astype(jnp.float32)
    same = (bi == bj)
    beats = same & ((sj > si) | ((sj == si) & (jj < ii)))
    rank_acc[...] += jnp.sum(beats.astype(jnp.float32), axis=1,
                             keepdims=True)
    cnt_acc[...] += jnp.sum(same.astype(jnp.float32), axis=1, keepdims=True)

    @pl.when(j == NJ - 1)
    def _fin():
        k = jnp.floor((4.0 * cnt_acc[...] + 4.0) * 0.2)
        keep_ref[...] = (rank_acc[...] < k).astype(jnp.float32)


def _rank_tc(score2d, batch_p):
    return pl.pallas_call(
        _rank_kern,
        grid=(NI, NJ),
        in_specs=[
            pl.BlockSpec((BI, 1), lambda i, j: (i, 0)),
            pl.BlockSpec((1, BJ), lambda i, j: (0, j)),
            pl.BlockSpec((BI, 1), lambda i, j: (i, 0)),
            pl.BlockSpec((1, BJ), lambda i, j: (0, j)),
        ],
        out_specs=pl.BlockSpec((BI, 1), lambda i, j: (i, 0)),
        out_shape=jax.ShapeDtypeStruct((NP, 1), jnp.float32),
        scratch_shapes=[
            pltpu.VMEM((BI, 1), jnp.float32),
            pltpu.VMEM((BI, 1), jnp.float32),
        ],
    )(score2d, score2d.reshape(1, NP), batch_p, batch_p.reshape(1, NP))


def _pool_kern(h_ref, s_ref, k_ref, b_ref, w1_ref, b1_ref, w2_ref, b2_ref,
               w3_ref, b3_ref, o_ref, gmp_acc, gap_acc, cnt_acc):
    i = pl.program_id(0)

    @pl.when(i == 0)
    def _init():
        gmp_acc[...] = jnp.full_like(gmp_acc, -jnp.inf)
        gap_acc[...] = jnp.zeros_like(gap_acc)
        cnt_acc[...] = jnp.zeros_like(cnt_acc)

    h = h_ref[...]
    sc = s_ref[...]
    kp = k_ref[...]
    bt = b_ref[...]
    val = h * sc
    g_iota = jax.lax.broadcasted_iota(jnp.int32, (1, 128), 1)
    onehot = ((bt == g_iota) & (kp > 0.0)).astype(jnp.float32)
    valk = val * kp
    gap_acc[...] += jax.lax.dot_general(
        onehot, valk, (((0,), (0,)), ((), ())),
        preferred_element_type=jnp.float32)
    cnt_acc[...] += jax.lax.dot_general(
        onehot, kp, (((0,), (0,)), ((), ())),
        preferred_element_type=jnp.float32)
    for g in range(G):
        mg = (bt == g) & (kp > 0.0)
        masked = jnp.where(mg, val, jnp.float32(-1e30))
        gmp_acc[pl.ds(g, 1), :] = jnp.maximum(
            gmp_acc[pl.ds(g, 1), :],
            jnp.max(masked, axis=0, keepdims=True))

    @pl.when(i == NI - 1)
    def _fin():
        gmp = gmp_acc[...]
        gap = gap_acc[pl.ds(0, G), :] / (cnt_acc[pl.ds(0, G), :] + 1e-16)
        cont = jnp.concatenate([gmp, gap], axis=1)
        z = jnp.maximum(jnp.dot(cont, w1_ref[...],
                                preferred_element_type=jnp.float32)
                        + b1_ref[...], 0.0)
        z = jnp.maximum(jnp.dot(z, w2_ref[...],
                                preferred_element_type=jnp.float32)
                        + b2_ref[...], 0.0)
        z = jnp.dot(z, w3_ref[...],
                    preferred_element_type=jnp.float32) + b3_ref[...]
        zmax = jnp.max(z, axis=1, keepdims=True)
        zs = z - zmax
        o_ref[...] = zs - jnp.log(jnp.sum(jnp.exp(zs), axis=1,
                                          keepdims=True))


def _pool_mlp_tc(h_full, score2d, keep2d, batch_p, W1_p, b1, W2, b2, W3p,
                 b3p):
    return pl.pallas_call(
        _pool_kern,
        grid=(NI,),
        in_specs=[
            pl.BlockSpec((BI, HC), lambda i: (i, 0)),
            pl.BlockSpec((BI, 1), lambda i: (i, 0)),
            pl.BlockSpec((BI, 1), lambda i: (i, 0)),
            pl.BlockSpec((BI, 1), lambda i: (i, 0)),
            pl.BlockSpec((2 * HC, 128), lambda i: (0, 0)),
            pl.BlockSpec((1, 128), lambda i: (0, 0)),
            pl.BlockSpec((128, 64), lambda i: (0, 0)),
            pl.BlockSpec((1, 64), lambda i: (0, 0)),
            pl.BlockSpec((64, 128), lambda i: (0, 0)),
            pl.BlockSpec((1, 128), lambda i: (0, 0)),
        ],
        out_specs=pl.BlockSpec((G, 128), lambda i: (0, 0)),
        out_shape=jax.ShapeDtypeStruct((G, 128), jnp.float32),
        scratch_shapes=[
            pltpu.VMEM((G, HC), jnp.float32),
            pltpu.VMEM((128, HC), jnp.float32),
            pltpu.VMEM((128, 1), jnp.float32),
        ],
    )(h_full, score2d, keep2d, batch_p, W1_p, b1, W2, b2, W3p, b3p)


# speedup vs baseline: 8.3558x; 1.1285x over previous
import jax
import jax.numpy as jnp
from jax import lax
from jax.experimental import pallas as pl
from jax.experimental.pallas import tpu as pltpu
from jax.experimental.pallas import tpu_sc as plsc

N = 10000
NP = 10240
E = 160000
D_IN = 128
D_EDGE = 16
H = 8
C = 64
HC = H * C
G = 16
RATIO = 0.8
OUT = 10
NEG = 0.2

CHUNK = 512
NCHUNK = 21
EP = 16 * NCHUNK * CHUNK
EFR = 160768
XR0 = 10240
EF0 = 2 * 10240
TR = EF0 + EFR
RANGE = 640
NROUND = NP // (2 * RANGE)
TROWS = RANGE // 16
FB = 8
B = 16
SELCAP = NCHUNK * CHUNK + 4 * B


def _perm_cols(W):
    return W.reshape(-1, H, C).transpose(0, 2, 1).reshape(-1, HC)


def _perm_vec(v):
    return v.reshape(H, C).T.reshape(HC)


def _perm_rows(M):
    return M.reshape(H, C, -1).transpose(1, 0, 2).reshape(HC, -1)


def _bcast_swap8(v):
    idx = jax.lax.iota(jnp.int32, 16) ^ 8
    dnums = lax.GatherDimensionNumbers(
        offset_dims=(), collapsed_slice_dims=(0,), start_index_map=(0,))
    return lax.gather(v, idx[:, None], dnums, (1,),
                      mode=lax.GatherScatterMode.PROMISE_IN_BOUNDS)


def _edge_body(tab, src3, attA, biasA, h_out,
               idx_c, sel_src, sel_dstl, sel_efi,
               idx48, dstl16, idx48b,
               rows48, rows48b,
               numb0, numb1, numb2, numb3, numb4,
               attv, biasv,
               sh0, sh1, sh2, sh3, sh4,
               sem0, sem1, sem2, sem0b):
    c = lax.axis_index("c")
    s = lax.axis_index("s")
    numb = (numb0, numb1, numb2, numb3, numb4)
    sh = (sh0, sh1, sh2, sh3, sh4)
    pltpu.sync_copy(attA, attv)
    pltpu.sync_copy(biasA, biasv)
    zf16 = jnp.zeros((16,), jnp.float32)
    zi16 = jnp.zeros((16,), jnp.int32)

    def _round(r, _0):
        lo = (2 * r + c) * RANGE

        def _zrow(i, _):
            for g in range(5):
                for j in range(8):
                    numb[g][i, pl.ds(j * 16, 16)] = zf16
            return 0
        lax.fori_loop(0, B, _zrow, 0)

        def _zcopy(f, _):
            cps = [pltpu.async_copy(
                numb[g].at[pl.ds(0, FB)],
                sh[g].at[pl.ds(s * TROWS + f * FB, FB)], sem1)
                for g in range(5)]
            for cp in cps:
                cp.wait()
            return 0
        lax.fori_loop(0, TROWS // FB, _zcopy, 0)
        plsc.subcore_barrier()

        def _chunk(ch, cnt):
            base = (s * NCHUNK + ch) * (3 * CHUNK)
            pltpu.sync_copy(src3.at[pl.ds(base, 3 * CHUNK)], idx_c)
            for v in range(CHUNK // 16):
                d = idx_c[pl.ds(CHUNK + v * 16, 16)]
                m = (d >= lo) & (d < lo + RANGE)
                plsc.store_compressed(sel_src.at[pl.ds(cnt, 16)],
                                      idx_c[pl.ds(v * 16, 16)], mask=m)
                plsc.store_compressed(sel_dstl.at[pl.ds(cnt, 16)], d - lo,
                                      mask=m)
                plsc.store_compressed(sel_efi.at[pl.ds(cnt, 16)],
                                      idx_c[pl.ds(2 * CHUNK + v * 16, 16)],
                                      mask=m)
                cnt = cnt + jnp.sum(m.astype(jnp.int32))
            return cnt
        cnt = lax.fori_loop(0, NCHUNK, _chunk, jnp.int32(0))
        for t in (0, 16, 32, 48):
            sel_src[pl.ds(cnt + t, 16)] = zi16
            sel_dstl[pl.ds(cnt + t, 16)] = zi16
            sel_efi[pl.ds(cnt + t, 16)] = zi16

        nsub = (cnt + B - 1) // B
        npair = (nsub + 1) // 2
        idx48s = (idx48, idx48b)
        rows48s = (rows48, rows48b)
        gsem = (sem0, sem0b)

        def _build_idx(slot, par):
            b0 = slot * B
            idx48s[par][pl.ds(0, 16)] = sel_src[pl.ds(b0, 16)]
            idx48s[par][pl.ds(16, 16)] = (sel_dstl[pl.ds(b0, 16)]
                                          + (lo + XR0))
            idx48s[par][pl.ds(32, 16)] = sel_efi[pl.ds(b0, 16)] + EF0

        for par in (0, 1):
            _build_idx(jnp.int32(par), par)
            pltpu.async_copy(tab.at[idx48s[par]], rows48s[par], gsem[par])

        def _pair(p, _):
            for par in (0, 1):
                slot = 2 * p + par
                b0 = slot * B
                pltpu.make_async_copy(tab.at[idx48s[par]], rows48s[par],
                                      gsem[par]).wait()
                rws = rows48s[par]
                nmb = numb

                def _edge(i, _):
                    acc = zf16
                    for j in range(HC // 16):
                        jsl = pl.ds(j * 16, 16)
                        z = (rws[i, jsl] + rws[i + 16, jsl]
                             + rws[i + 32, jsl])
                        z = jnp.where(z >= 0.0, z, z * NEG)
                        acc = acc + z * attv[jsl]
                    hs = acc + _bcast_swap8(acc)
                    valid = ((b0 + i) < cnt).astype(jnp.float32)
                    ex = jnp.exp(hs) * valid
                    nmb[4][i, pl.ds(0, 16)] = ex
                    for j in range(HC // 16):
                        jsl = pl.ds(j * 16, 16)
                        nmb[j // 8][i, pl.ds((j % 8) * 16, 16)] = (
                            rws[i, jsl] * ex)
                    return 0
                lax.fori_loop(0, B, _edge, 0)
                dstl16[pl.ds(0, 16)] = sel_dstl[pl.ds(b0, 16)]
                cps = [pltpu.async_copy(nmb[g], sh[g].at[dstl16], sem2,
                                        add=True) for g in range(5)]
                for cp in cps:
                    cp.wait()
                _build_idx(slot + 2, par)
                pltpu.async_copy(tab.at[idx48s[par]], rows48s[par],
                                 gsem[par])
            return 0
        lax.fori_loop(0, npair, _pair, 0)
        for par in (0, 1):
            pltpu.make_async_copy(tab.at[idx48s[par]], rows48s[par],
                                  gsem[par]).wait()
        plsc.subcore_barrier()

        def _flush(f, _):
            r0 = s * TROWS + f * FB
            cps = [pltpu.async_copy(sh[g].at[pl.ds(r0, FB)],
                                    numb[g].at[pl.ds(0, FB)], sem1)
                   for g in range(5)]
            for cp in cps:
                cp.wait()

            def _row(row, _):
                rec = 1.0 / (numb4[row, pl.ds(0, 16)] + 1e-30)
                for j in range(HC // 16):
                    jsl = pl.ds(j * 16, 16)
                    rows48[row, jsl] = jnp.maximum(
                        numb[j // 8][row, pl.ds((j % 8) * 16, 16)] * rec
                        + biasv[jsl], 0.0)
                return 0
            lax.fori_loop(0, FB, _row, 0)
            pltpu.sync_copy(rows48.at[pl.ds(0, FB)],
                            h_out.at[pl.ds(lo + r0, FB)])
            return 0
        lax.fori_loop(0, TROWS // FB, _flush, 0)
        plsc.subcore_barrier()
        return 0

    lax.fori_loop(0, NROUND, _round, 0)


def _gat_conv_sc(tab, src3, att_p, bias_p):
    mesh = plsc.VectorSubcoreMesh(core_axis_name="c", subcore_axis_name="s")
    f32 = jnp.float32
    i32 = jnp.int32
    return pl.kernel(
        _edge_body,
        out_type=jax.ShapeDtypeStruct((NP, HC), f32),
        mesh=mesh,
        compiler_params=pltpu.CompilerParams(needs_layout_passes=False),
        scratch_types=[
            pltpu.VMEM((3 * CHUNK,), i32),
            pltpu.VMEM((SELCAP,), i32),
            pltpu.VMEM((SELCAP,), i32),
            pltpu.VMEM((SELCAP,), i32),
            pltpu.VMEM((3 * B,), i32),
            pltpu.VMEM((16,), i32),
            pltpu.VMEM((3 * B,), i32),
            pltpu.VMEM((3 * B, HC), f32),
            pltpu.VMEM((3 * B, HC), f32),
            pltpu.VMEM((B, 128), f32),
            pltpu.VMEM((B, 128), f32),
            pltpu.VMEM((B, 128), f32),
            pltpu.VMEM((B, 128), f32),
            pltpu.VMEM((B, 128), f32),
            pltpu.VMEM((HC,), f32),
            pltpu.VMEM((HC,), f32),
            pltpu.VMEM_SHARED((RANGE, 128), f32),
            pltpu.VMEM_SHARED((RANGE, 128), f32),
            pltpu.VMEM_SHARED((RANGE, 128), f32),
            pltpu.VMEM_SHARED((RANGE, 128), f32),
            pltpu.VMEM_SHARED((RANGE, 128), f32),
            pltpu.SemaphoreType.DMA,
            pltpu.SemaphoreType.DMA,
            pltpu.SemaphoreType.DMA,
            pltpu.SemaphoreType.DMA,
        ],
    )(tab, src3, att_p, bias_p)


def _fused_proj_kern(x_ref, ea_ref, wl_ref, wr_ref, we_ref, bl_ref, br_ref,
                     o_ref):
    i = pl.program_id(0)

    @pl.when(i < 10)
    def _xl():
        o_ref[...] = jnp.dot(x_ref[...], wl_ref[...],
                             preferred_element_type=jnp.float32) + bl_ref[...]

    @pl.when((i >= 10) & (i < 20))
    def _xr():
        o_ref[...] = jnp.dot(x_ref[...], wr_ref[...],
                             preferred_element_type=jnp.float32) + br_ref[...]

    @pl.when(i >= 20)
    def _ef():
        o_ref[...] = jnp.dot(ea_ref[...], we_ref[...],
                             preferred_element_type=jnp.float32)


def _fused_proj(x_pad, ea_pad, W_lp, W_rp, W_ep, b_lp, b_rp):
    nb = TR // 1024
    return pl.pallas_call(
        _fused_proj_kern,
        grid=(nb,),
        in_specs=[
            pl.BlockSpec((1024, D_IN),
                         lambda i: (jnp.where(i < 10, i,
                                              jnp.where(i < 20, i - 10, 0)),
                                    0)),
            pl.BlockSpec((1024, D_EDGE),
                         lambda i: (jnp.where(i >= 20, i - 20, 0), 0)),
            pl.BlockSpec((D_IN, HC), lambda i: (0, 0)),
            pl.BlockSpec((D_IN, HC), lambda i: (0, 0)),
            pl.BlockSpec((D_EDGE, HC), lambda i: (0, 0)),
            pl.BlockSpec((1, HC), lambda i: (0, 0)),
            pl.BlockSpec((1, HC), lambda i: (0, 0)),
        ],
        out_specs=pl.BlockSpec((1024, HC), lambda i: (i, 0)),
        out_shape=jax.ShapeDtypeStruct((TR, HC), jnp.float32),
    )(x_pad, ea_pad, W_lp, W_rp, W_ep, b_lp[None, :], b_rp[None, :])


def kernel(x, edge_index, edge_attr, batch, W_l, b_l, W_r, b_r, W_e, att,
           conv_bias, pool_p, W1, b1, W2, b2, W3, b3):
    n = N
    loop = jnp.arange(n, dtype=edge_index.dtype)
    pad = EP - (E + n)
    src_all = jnp.concatenate(
        [edge_index[0], loop, jnp.zeros((pad,), jnp.int32)])
    dst_s = jnp.concatenate(
        [edge_index[1], loop, jnp.full((pad,), 1 << 30, jnp.int32)])
    efi = jnp.concatenate(
        [jnp.arange(E, dtype=jnp.int32), jnp.full((n,), E, jnp.int32),
         jnp.zeros((pad,), jnp.int32)])
    src3 = (jnp.stack([src_all, dst_s, efi], axis=0)
            .reshape(3, EP // CHUNK, CHUNK).transpose(1, 0, 2).reshape(-1))

    W_lp, b_lp = _perm_cols(W_l), _perm_vec(b_l)
    W_rp, b_rp = _perm_cols(W_r), _perm_vec(b_r)
    W_ep = _perm_cols(W_e)
    att_p = att.T.reshape(HC)
    bias_p = _perm_vec(conv_bias)
    pool_pp = _perm_vec(pool_p)

    x_pad = jnp.pad(x, ((0, NP - N), (0, 0)))
    ea_mean = jnp.mean(edge_attr, axis=0, keepdims=True)
    ea_pad = jnp.concatenate(
        [edge_attr, jnp.tile(ea_mean, (EFR - E, 1))], axis=0)
    tab = _fused_proj(x_pad, ea_pad, W_lp, W_rp, W_ep, b_lp, b_rp)

    h_full = _gat_conv_sc(tab, src3, att_p, bias_p)

    score2d = _score_tc(h_full, pool_pp)
    batch_p = jnp.concatenate(
        [batch, jnp.full((NP - N,), 1 << 20, jnp.int32)]).reshape(NP, 1)
    keep2d = _rank_tc(score2d, batch_p)
    W1_p = jnp.concatenate([_perm_rows(W1[:HC]), _perm_rows(W1[HC:])], axis=0)
    W3p = jnp.pad(W3, ((0, 0), (0, 128 - OUT)))
    b3p = jnp.pad(b3, (0, 128 - OUT), constant_values=-1e30)
    out = _pool_mlp_tc(h_full, score2d, keep2d, batch_p,
                       W1_p, b1.reshape(1, -1), W2, b2.reshape(1, -1),
                       W3p, b3p.reshape(1, -1))
    return out[:, :OUT]


BI = 256
BJ = 2048
NI = NP // BI
NJ = NP // BJ


def _score_kern(h_ref, p_ref, o_ref):
    p = p_ref[...]
    norm = jnp.sqrt(jnp.sum(p * p))
    o_ref[...] = jnp.tanh(
        jnp.dot(h_ref[...], p, preferred_element_type=jnp.float32) / norm)


def _score_tc(h_full, pool_pp):
    return pl.pallas_call(
        _score_kern,
        grid=(NI,),
        in_specs=[
            pl.BlockSpec((BI, HC), lambda i: (i, 0)),
            pl.BlockSpec((HC, 1), lambda i: (0, 0)),
        ],
        out_specs=pl.BlockSpec((BI, 1), lambda i: (i, 0)),
        out_shape=jax.ShapeDtypeStruct((NP, 1), jnp.float32),
    )(h_full, pool_pp.reshape(HC, 1))


def _rank_kern(si_ref, sj_ref, bi_ref, bj_ref, keep_ref, rank_acc, cnt_acc):
    i = pl.program_id(0)
    j = pl.program_id(1)

    @pl.when(j == 0)
    def _init():
        rank_acc[...] = jnp.zeros_like(rank_acc)
        cnt_acc[...] = jnp.zeros_like(cnt_acc)

    si = si_ref[...]
    sj = sj_ref[...]
    bi = bi_ref[...]
    bj = bj_ref[...]
    ii = (jax.lax.broadcasted_iota(jnp.int32, (BI, 1), 0)
          + i * BI).astype(jnp.float32)
    jj = (jax.lax.broadcasted_iota(jnp.int32, (1, BJ), 1)
          + j * BJ).astype(jnp.float32)
    same = (bi == bj)
    beats = same & ((sj > si) | ((sj == si) & (jj < ii)))
    rank_acc[...] += jnp.sum(beats.astype(jnp.float32), axis=1,
                             keepdims=True)
    cnt_acc[...] += jnp.sum(same.astype(jnp.float32), axis=1, keepdims=True)

    @pl.when(j == NJ - 1)
    def _fin():
        k = jnp.floor((4.0 * cnt_acc[...] + 4.0) * 0.2)
        keep_ref[...] = (rank_acc[...] < k).astype(jnp.float32)


def _rank_tc(score2d, batch_p):
    return pl.pallas_call(
        _rank_kern,
        grid=(NI, NJ),
        in_specs=[
            pl.BlockSpec((BI, 1), lambda i, j: (i, 0)),
            pl.BlockSpec((1, BJ), lambda i, j: (0, j)),
            pl.BlockSpec((BI, 1), lambda i, j: (i, 0)),
            pl.BlockSpec((1, BJ), lambda i, j: (0, j)),
        ],
        out_specs=pl.BlockSpec((BI, 1), lambda i, j: (i, 0)),
        out_shape=jax.ShapeDtypeStruct((NP, 1), jnp.float32),
        scratch_shapes=[
            pltpu.VMEM((BI, 1), jnp.float32),
            pltpu.VMEM((BI, 1), jnp.float32),
        ],
    )(score2d, score2d.reshape(1, NP), batch_p, batch_p.reshape(1, NP))


def _pool_kern(h_ref, s_ref, k_ref, b_ref, w1_ref, b1_ref, w2_ref, b2_ref,
               w3_ref, b3_ref, o_ref, gmp_acc, gap_acc, cnt_acc):
    i = pl.program_id(0)

    @pl.when(i == 0)
    def _init():
        gmp_acc[...] = jnp.full_like(gmp_acc, -jnp.inf)
        gap_acc[...] = jnp.zeros_like(gap_acc)
        cnt_acc[...] = jnp.zeros_like(cnt_acc)

    h = h_ref[...]
    sc = s_ref[...]
    kp = k_ref[...]
    bt = b_ref[...]
    val = h * sc
    g_iota = jax.lax.broadcasted_iota(jnp.int32, (1, 128), 1)
    onehot = ((bt == g_iota) & (kp > 0.0)).astype(jnp.float32)
    valk = val * kp
    gap_acc[...] += jax.lax.dot_general(
        onehot, valk, (((0,), (0,)), ((), ())),
        preferred_element_type=jnp.float32)
    cnt_acc[...] += jax.lax.dot_general(
        onehot, kp, (((0,), (0,)), ((), ())),
        preferred_element_type=jnp.float32)
    for g in range(G):
        mg = (bt == g) & (kp > 0.0)
        masked = jnp.where(mg, val, jnp.float32(-1e30))
        gmp_acc[pl.ds(g, 1), :] = jnp.maximum(
            gmp_acc[pl.ds(g, 1), :],
            jnp.max(masked, axis=0, keepdims=True))

    @pl.when(i == NI - 1)
    def _fin():
        gmp = gmp_acc[...]
        gap = gap_acc[pl.ds(0, G), :] / (cnt_acc[pl.ds(0, G), :] + 1e-16)
        cont = jnp.concatenate([gmp, gap], axis=1)
        z = jnp.maximum(jnp.dot(cont, w1_ref[...],
                                preferred_element_type=jnp.float32)
                        + b1_ref[...], 0.0)
        z = jnp.maximum(jnp.dot(z, w2_ref[...],
                                preferred_element_type=jnp.float32)
                        + b2_ref[...], 0.0)
        z = jnp.dot(z, w3_ref[...],
                    preferred_element_type=jnp.float32) + b3_ref[...]
        zmax = jnp.max(z, axis=1, keepdims=True)
        zs = z - zmax
        o_ref[...] = zs - jnp.log(jnp.sum(jnp.exp(zs), axis=1,
                                          keepdims=True))


def _pool_mlp_tc(h_full, score2d, keep2d, batch_p, W1_p, b1, W2, b2, W3p,
                 b3p):
    return pl.pallas_call(
        _pool_kern,
        grid=(NI,),
        in_specs=[
            pl.BlockSpec((BI, HC), lambda i: (i, 0)),
            pl.BlockSpec((BI, 1), lambda i: (i, 0)),
            pl.BlockSpec((BI, 1), lambda i: (i, 0)),
            pl.BlockSpec((BI, 1), lambda i: (i, 0)),
            pl.BlockSpec((2 * HC, 128), lambda i: (0, 0)),
            pl.BlockSpec((1, 128), lambda i: (0, 0)),
            pl.BlockSpec((128, 64), lambda i: (0, 0)),
            pl.BlockSpec((1, 64), lambda i: (0, 0)),
            pl.BlockSpec((64, 128), lambda i: (0, 0)),
            pl.BlockSpec((1, 128), lambda i: (0, 0)),
        ],
        out_specs=pl.BlockSpec((G, 128), lambda i: (0, 0)),
        out_shape=jax.ShapeDtypeStruct((G, 128), jnp.float32),
        scratch_shapes=[
            pltpu.VMEM((G, HC), jnp.float32),
            pltpu.VMEM((128, HC), jnp.float32),
            pltpu.VMEM((128, 1), jnp.float32),
        ],
    )(h_full, score2d, keep2d, batch_p, W1_p, b1, W2, b2, W3p, b3p)


# speedup vs baseline: 8.5532x; 1.0236x over previous
import jax
import jax.numpy as jnp
from jax import lax
from jax.experimental import pallas as pl
from jax.experimental.pallas import tpu as pltpu
from jax.experimental.pallas import tpu_sc as plsc

N = 10000
NP = 10240
E = 160000
D_IN = 128
D_EDGE = 16
H = 8
C = 64
HC = H * C
G = 16
RATIO = 0.8
OUT = 10
NEG = 0.2

CHUNK = 512
NCHUNK = 21
EP = 16 * NCHUNK * CHUNK
EFR = 160768
XR0 = 10240
EF0 = 2 * 10240
TR = EF0 + EFR
RANGE = 640
NROUND = NP // (2 * RANGE)
TROWS = RANGE // 16
FB = 8
B = 16
SELCAP = NCHUNK * CHUNK + 4 * B


def _perm_cols(W):
    return W.reshape(-1, H, C).transpose(0, 2, 1).reshape(-1, HC)


def _perm_vec(v):
    return v.reshape(H, C).T.reshape(HC)


def _perm_rows(M):
    return M.reshape(H, C, -1).transpose(1, 0, 2).reshape(HC, -1)


def _bcast_swap8(v):
    idx = jax.lax.iota(jnp.int32, 16) ^ 8
    dnums = lax.GatherDimensionNumbers(
        offset_dims=(), collapsed_slice_dims=(0,), start_index_map=(0,))
    return lax.gather(v, idx[:, None], dnums, (1,),
                      mode=lax.GatherScatterMode.PROMISE_IN_BOUNDS)


def _edge_body(tab, src3, attA, biasA, h_out,
               idx_c, idx_cb, sel_src, sel_dstl, sel_efi,
               idx48, dstl16, idx48b,
               rows48, rows48b,
               numb0, numb1, numb2, numb3, numb4,
               attv, biasv,
               sh0, sh1, sh2, sh3, sh4,
               sem0, sem1, sem2, sem0b, sem1b):
    c = lax.axis_index("c")
    s = lax.axis_index("s")
    numb = (numb0, numb1, numb2, numb3, numb4)
    idx_cs = (idx_c, idx_cb)
    gsemA = (sem1, sem1b)
    sh = (sh0, sh1, sh2, sh3, sh4)
    pltpu.sync_copy(attA, attv)
    pltpu.sync_copy(biasA, biasv)
    zf16 = jnp.zeros((16,), jnp.float32)
    zi16 = jnp.zeros((16,), jnp.int32)

    def _round(r, _0):
        lo = (2 * r + c) * RANGE

        def _zrow(i, _):
            for g in range(5):
                for j in range(8):
                    numb[g][i, pl.ds(j * 16, 16)] = zf16
            return 0
        lax.fori_loop(0, B, _zrow, 0)

        def _zcopy(f, _):
            cps = [pltpu.async_copy(
                numb[g].at[pl.ds(0, FB)],
                sh[g].at[pl.ds(s * TROWS + f * FB, FB)], sem1)
                for g in range(5)]
            for cp in cps:
                cp.wait()
            return 0
        lax.fori_loop(0, TROWS // FB, _zcopy, 0)
        plsc.subcore_barrier()

        def _start_chunk(ch, par):
            base = (s * NCHUNK + ch) * (3 * CHUNK)
            pltpu.async_copy(src3.at[pl.ds(base, 3 * CHUNK)],
                             idx_cs[par], gsemA[par])

        def _wait_chunk(ch, par):
            base = (s * NCHUNK + ch) * (3 * CHUNK)
            pltpu.make_async_copy(src3.at[pl.ds(base, 3 * CHUNK)],
                                  idx_cs[par], gsemA[par]).wait()

        for par in (0, 1):
            _start_chunk(jnp.int32(par), par)

        def _chunkpair(pch, cnt):
            for par in (0, 1):
                ch = 2 * pch + par
                _wait_chunk(ch, par)
                idc = idx_cs[par]
                ok = ch < NCHUNK
                for v in range(CHUNK // 16):
                    d = idc[pl.ds(CHUNK + v * 16, 16)]
                    m = (d >= lo) & (d < lo + RANGE) & ok
                    plsc.store_compressed(sel_src.at[pl.ds(cnt, 16)],
                                          idc[pl.ds(v * 16, 16)], mask=m)
                    plsc.store_compressed(sel_dstl.at[pl.ds(cnt, 16)],
                                          d - lo, mask=m)
                    plsc.store_compressed(
                        sel_efi.at[pl.ds(cnt, 16)],
                        idc[pl.ds(2 * CHUNK + v * 16, 16)], mask=m)
                    cnt = cnt + jnp.sum(m.astype(jnp.int32))
                _start_chunk(jnp.minimum(ch + 2, NCHUNK + 1), par)
            return cnt
        cnt = lax.fori_loop(0, (NCHUNK + 2) // 2, _chunkpair, jnp.int32(0))
        for par in (0, 1):
            _wait_chunk(jnp.int32(NCHUNK + 1), par)
        for t in (0, 16, 32, 48):
            sel_src[pl.ds(cnt + t, 16)] = zi16
            sel_dstl[pl.ds(cnt + t, 16)] = zi16
            sel_efi[pl.ds(cnt + t, 16)] = zi16

        nsub = (cnt + B - 1) // B
        npair = (nsub + 1) // 2
        idx48s = (idx48, idx48b)
        rows48s = (rows48, rows48b)
        gsem = (sem0, sem0b)

        def _build_idx(slot, par):
            b0 = slot * B
            idx48s[par][pl.ds(0, 16)] = sel_src[pl.ds(b0, 16)]
            idx48s[par][pl.ds(16, 16)] = (sel_dstl[pl.ds(b0, 16)]
                                          + (lo + XR0))
            idx48s[par][pl.ds(32, 16)] = sel_efi[pl.ds(b0, 16)] + EF0

        for par in (0, 1):
            _build_idx(jnp.int32(par), par)
            pltpu.async_copy(tab.at[idx48s[par]], rows48s[par], gsem[par])

        def _pair(p, _):
            for par in (0, 1):
                slot = 2 * p + par
                b0 = slot * B
                pltpu.make_async_copy(tab.at[idx48s[par]], rows48s[par],
                                      gsem[par]).wait()
                rws = rows48s[par]
                nmb = numb

                def _edge(i, _):
                    acc = zf16
                    for j in range(HC // 16):
                        jsl = pl.ds(j * 16, 16)
                        z = (rws[i, jsl] + rws[i + 16, jsl]
                             + rws[i + 32, jsl])
                        z = jnp.where(z >= 0.0, z, z * NEG)
                        acc = acc + z * attv[jsl]
                    hs = acc + _bcast_swap8(acc)
                    valid = ((b0 + i) < cnt).astype(jnp.float32)
                    ex = jnp.exp(hs) * valid
                    nmb[4][i, pl.ds(0, 16)] = ex
                    for j in range(HC // 16):
                        jsl = pl.ds(j * 16, 16)
                        nmb[j // 8][i, pl.ds((j % 8) * 16, 16)] = (
                            rws[i, jsl] * ex)
                    return 0
                lax.fori_loop(0, B, _edge, 0)
                dstl16[pl.ds(0, 16)] = sel_dstl[pl.ds(b0, 16)]
                cps = [pltpu.async_copy(nmb[g], sh[g].at[dstl16], sem2,
                                        add=True) for g in range(5)]
                for cp in cps:
                    cp.wait()
                _build_idx(slot + 2, par)
                pltpu.async_copy(tab.at[idx48s[par]], rows48s[par],
                                 gsem[par])
            return 0
        lax.fori_loop(0, npair, _pair, 0)
        for par in (0, 1):
            pltpu.make_async_copy(tab.at[idx48s[par]], rows48s[par],
                                  gsem[par]).wait()
        plsc.subcore_barrier()

        def _flush(f, _):
            r0 = s * TROWS + f * FB
            cps = [pltpu.async_copy(sh[g].at[pl.ds(r0, FB)],
                                    numb[g].at[pl.ds(0, FB)], sem1)
                   for g in range(5)]
            for cp in cps:
                cp.wait()

            def _row(row, _):
                rec = 1.0 / (numb4[row, pl.ds(0, 16)] + 1e-30)
                for j in range(HC // 16):
                    jsl = pl.ds(j * 16, 16)
                    rows48[row, jsl] = jnp.maximum(
                        numb[j // 8][row, pl.ds((j % 8) * 16, 16)] * rec
                        + biasv[jsl], 0.0)
                return 0
            lax.fori_loop(0, FB, _row, 0)
            pltpu.sync_copy(rows48.at[pl.ds(0, FB)],
                            h_out.at[pl.ds(lo + r0, FB)])
            return 0
        lax.fori_loop(0, TROWS // FB, _flush, 0)
        plsc.subcore_barrier()
        return 0

    lax.fori_loop(0, NROUND, _round, 0)


def _gat_conv_sc(tab, src3, att_p, bias_p):
    mesh = plsc.VectorSubcoreMesh(core_axis_name="c", subcore_axis_name="s")
    f32 = jnp.float32
    i32 = jnp.int32
    return pl.kernel(
        _edge_body,
        out_type=jax.ShapeDtypeStruct((NP, HC), f32),
        mesh=mesh,
        compiler_params=pltpu.CompilerParams(needs_layout_passes=False),
        scratch_types=[
            pltpu.VMEM((3 * CHUNK,), i32),
            pltpu.VMEM((3 * CHUNK,), i32),
            pltpu.VMEM((SELCAP,), i32),
            pltpu.VMEM((SELCAP,), i32),
            pltpu.VMEM((SELCAP,), i32),
            pltpu.VMEM((3 * B,), i32),
            pltpu.VMEM((16,), i32),
            pltpu.VMEM((3 * B,), i32),
            pltpu.VMEM((3 * B, HC), f32),
            pltpu.VMEM((3 * B, HC), f32),
            pltpu.VMEM((B, 128), f32),
            pltpu.VMEM((B, 128), f32),
            pltpu.VMEM((B, 128), f32),
            pltpu.VMEM((B, 128), f32),
            pltpu.VMEM((B, 128), f32),
            pltpu.VMEM((HC,), f32),
            pltpu.VMEM((HC,), f32),
            pltpu.VMEM_SHARED((RANGE, 128), f32),
            pltpu.VMEM_SHARED((RANGE, 128), f32),
            pltpu.VMEM_SHARED((RANGE, 128), f32),
            pltpu.VMEM_SHARED((RANGE, 128), f32),
            pltpu.VMEM_SHARED((RANGE, 128), f32),
            pltpu.SemaphoreType.DMA,
            pltpu.SemaphoreType.DMA,
            pltpu.SemaphoreType.DMA,
            pltpu.SemaphoreType.DMA,
            pltpu.SemaphoreType.DMA,
        ],
    )(tab, src3, att_p, bias_p)


def _fused_proj_kern(x_ref, ea_ref, wl_ref, wr_ref, we_ref, bl_ref, br_ref,
                     o_ref):
    i = pl.program_id(0)

    @pl.when(i < 10)
    def _xl():
        o_ref[...] = jnp.dot(x_ref[...], wl_ref[...],
                             preferred_element_type=jnp.float32) + bl_ref[...]

    @pl.when((i >= 10) & (i < 20))
    def _xr():
        o_ref[...] = jnp.dot(x_ref[...], wr_ref[...],
                             preferred_element_type=jnp.float32) + br_ref[...]

    @pl.when(i >= 20)
    def _ef():
        o_ref[...] = jnp.dot(ea_ref[...], we_ref[...],
                             preferred_element_type=jnp.float32)


def _fused_proj(x_pad, ea_pad, W_lp, W_rp, W_ep, b_lp, b_rp):
    nb = TR // 1024
    return pl.pallas_call(
        _fused_proj_kern,
        grid=(nb,),
        in_specs=[
            pl.BlockSpec((1024, D_IN),
                         lambda i: (jnp.where(i < 10, i,
                                              jnp.where(i < 20, i - 10, 0)),
                                    0)),
            pl.BlockSpec((1024, D_EDGE),
                         lambda i: (jnp.where(i >= 20, i - 20, 0), 0)),
            pl.BlockSpec((D_IN, HC), lambda i: (0, 0)),
            pl.BlockSpec((D_IN, HC), lambda i: (0, 0)),
            pl.BlockSpec((D_EDGE, HC), lambda i: (0, 0)),
            pl.BlockSpec((1, HC), lambda i: (0, 0)),
            pl.BlockSpec((1, HC), lambda i: (0, 0)),
        ],
        out_specs=pl.BlockSpec((1024, HC), lambda i: (i, 0)),
        out_shape=jax.ShapeDtypeStruct((TR, HC), jnp.float32),
    )(x_pad, ea_pad, W_lp, W_rp, W_ep, b_lp[None, :], b_rp[None, :])


def kernel(x, edge_index, edge_attr, batch, W_l, b_l, W_r, b_r, W_e, att,
           conv_bias, pool_p, W1, b1, W2, b2, W3, b3):
    n = N
    loop = jnp.arange(n, dtype=edge_index.dtype)
    pad = EP - (E + n)
    src_all = jnp.concatenate(
        [edge_index[0], loop, jnp.zeros((pad,), jnp.int32)])
    dst_s = jnp.concatenate(
        [edge_index[1], loop, jnp.full((pad,), 1 << 30, jnp.int32)])
    efi = jnp.concatenate(
        [jnp.arange(E, dtype=jnp.int32), jnp.full((n,), E, jnp.int32),
         jnp.zeros((pad,), jnp.int32)])
    src3 = (jnp.stack([src_all, dst_s, efi], axis=0)
            .reshape(3, EP // CHUNK, CHUNK).transpose(1, 0, 2).reshape(-1))
    src3 = jnp.concatenate([src3, jnp.zeros((2 * 3 * CHUNK,), jnp.int32)])

    W_lp, b_lp = _perm_cols(W_l), _perm_vec(b_l)
    W_rp, b_rp = _perm_cols(W_r), _perm_vec(b_r)
    W_ep = _perm_cols(W_e)
    att_p = att.T.reshape(HC)
    bias_p = _perm_vec(conv_bias)
    pool_pp = _perm_vec(pool_p)

    x_pad = jnp.pad(x, ((0, NP - N), (0, 0)))
    ea_mean = jnp.mean(edge_attr, axis=0, keepdims=True)
    ea_pad = jnp.concatenate(
        [edge_attr, jnp.tile(ea_mean, (EFR - E, 1))], axis=0)
    tab = _fused_proj(x_pad, ea_pad, W_lp, W_rp, W_ep, b_lp, b_rp)

    h_full = _gat_conv_sc(tab, src3, att_p, bias_p)

    score2d = _score_tc(h_full, pool_pp)
    batch_p = jnp.concatenate(
        [batch, jnp.full((NP - N,), 1 << 20, jnp.int32)]).reshape(NP, 1)
    keep2d = _rank_tc(score2d, batch_p)
    W1_p = jnp.concatenate([_perm_rows(W1[:HC]), _perm_rows(W1[HC:])], axis=0)
    W3p = jnp.pad(W3, ((0, 0), (0, 128 - OUT)))
    b3p = jnp.pad(b3, (0, 128 - OUT), constant_values=-1e30)
    out = _pool_mlp_tc(h_full, score2d, keep2d, batch_p,
                       W1_p, b1.reshape(1, -1), W2, b2.reshape(1, -1),
                       W3p, b3p.reshape(1, -1))
    return out[:, :OUT]


BI = 256
BJ = 2048
NI = NP // BI
NJ = NP // BJ


def _score_kern(h_ref, p_ref, o_ref):
    p = p_ref[...]
    norm = jnp.sqrt(jnp.sum(p * p))
    o_ref[...] = jnp.tanh(
        jnp.dot(h_ref[...], p, preferred_element_type=jnp.float32) / norm)


def _score_tc(h_full, pool_pp):
    return pl.pallas_call(
        _score_kern,
        grid=(NI,),
        in_specs=[
            pl.BlockSpec((BI, HC), lambda i: (i, 0)),
            pl.BlockSpec((HC, 1), lambda i: (0, 0)),
        ],
        out_specs=pl.BlockSpec((BI, 1), lambda i: (i, 0)),
        out_shape=jax.ShapeDtypeStruct((NP, 1), jnp.float32),
    )(h_full, pool_pp.reshape(HC, 1))


def _rank_kern(si_ref, sj_ref, bi_ref, bj_ref, keep_ref, rank_acc, cnt_acc):
    i = pl.program_id(0)
    j = pl.program_id(1)

    @pl.when(j == 0)
    def _init():
        rank_acc[...] = jnp.zeros_like(rank_acc)
        cnt_acc[...] = jnp.zeros_like(cnt_acc)

    si = si_ref[...]
    sj = sj_ref[...]
    bi = bi_ref[...]
    bj = bj_ref[...]
    ii = (jax.lax.broadcasted_iota(jnp.int32, (BI, 1), 0)
          + i * BI).astype(jnp.float32)
    jj = (jax.lax.broadcasted_iota(jnp.int32, (1, BJ), 1)
          + j * BJ).astype(jnp.float32)
    same = (bi == bj)
    beats = same & ((sj > si) | ((sj == si) & (jj < ii)))
    rank_acc[...] += jnp.sum(beats.astype(jnp.float32), axis=1,
                             keepdims=True)
    cnt_acc[...] += jnp.sum(same.astype(jnp.float32), axis=1, keepdims=True)

    @pl.when(j == NJ - 1)
    def _fin():
        k = jnp.floor((4.0 * cnt_acc[...] + 4.0) * 0.2)
        keep_ref[...] = (rank_acc[...] < k).astype(jnp.float32)


def _rank_tc(score2d, batch_p):
    return pl.pallas_call(
        _rank_kern,
        grid=(NI, NJ),
        in_specs=[
            pl.BlockSpec((BI, 1), lambda i, j: (i, 0)),
            pl.BlockSpec((1, BJ), lambda i, j: (0, j)),
            pl.BlockSpec((BI, 1), lambda i, j: (i, 0)),
            pl.BlockSpec((1, BJ), lambda i, j: (0, j)),
        ],
        out_specs=pl.BlockSpec((BI, 1), lambda i, j: (i, 0)),
        out_shape=jax.ShapeDtypeStruct((NP, 1), jnp.float32),
        scratch_shapes=[
            pltpu.VMEM((BI, 1), jnp.float32),
            pltpu.VMEM((BI, 1), jnp.float32),
        ],
    )(score2d, score2d.reshape(1, NP), batch_p, batch_p.reshape(1, NP))


def _pool_kern(h_ref, s_ref, k_ref, b_ref, w1_ref, b1_ref, w2_ref, b2_ref,
               w3_ref, b3_ref, o_ref, gmp_acc, gap_acc, cnt_acc):
    i = pl.program_id(0)

    @pl.when(i == 0)
    def _init():
        gmp_acc[...] = jnp.full_like(gmp_acc, -jnp.inf)
        gap_acc[...] = jnp.zeros_like(gap_acc)
        cnt_acc[...] = jnp.zeros_like(cnt_acc)

    h = h_ref[...]
    sc = s_ref[...]
    kp = k_ref[...]
    bt = b_ref[...]
    val = h * sc
    g_iota = jax.lax.broadcasted_iota(jnp.int32, (1, 128), 1)
    onehot = ((bt == g_iota) & (kp > 0.0)).astype(jnp.float32)
    valk = val * kp
    gap_acc[...] += jax.lax.dot_general(
        onehot, valk, (((0,), (0,)), ((), ())),
        preferred_element_type=jnp.float32)
    cnt_acc[...] += jax.lax.dot_general(
        onehot, kp, (((0,), (0,)), ((), ())),
        preferred_element_type=jnp.float32)
    for g in range(G):
        mg = (bt == g) & (kp > 0.0)
        masked = jnp.where(mg, val, jnp.float32(-1e30))
        gmp_acc[pl.ds(g, 1), :] = jnp.maximum(
            gmp_acc[pl.ds(g, 1), :],
            jnp.max(masked, axis=0, keepdims=True))

    @pl.when(i == NI - 1)
    def _fin():
        gmp = gmp_acc[...]
        gap = gap_acc[pl.ds(0, G), :] / (cnt_acc[pl.ds(0, G), :] + 1e-16)
        cont = jnp.concatenate([gmp, gap], axis=1)
        z = jnp.maximum(jnp.dot(cont, w1_ref[...],
                                preferred_element_type=jnp.float32)
                        + b1_ref[...], 0.0)
        z = jnp.maximum(jnp.dot(z, w2_ref[...],
                                preferred_element_type=jnp.float32)
                        + b2_ref[...], 0.0)
        z = jnp.dot(z, w3_ref[...],
                    preferred_element_type=jnp.float32) + b3_ref[...]
        zmax = jnp.max(z, axis=1, keepdims=True)
        zs = z - zmax
        o_ref[...] = zs - jnp.log(jnp.sum(jnp.exp(zs), axis=1,
                                          keepdims=True))


def _pool_mlp_tc(h_full, score2d, keep2d, batch_p, W1_p, b1, W2, b2, W3p,
                 b3p):
    return pl.pallas_call(
        _pool_kern,
        grid=(NI,),
        in_specs=[
            pl.BlockSpec((BI, HC), lambda i: (i, 0)),
            pl.BlockSpec((BI, 1), lambda i: (i, 0)),
            pl.BlockSpec((BI, 1), lambda i: (i, 0)),
            pl.BlockSpec((BI, 1), lambda i: (i, 0)),
            pl.BlockSpec((2 * HC, 128), lambda i: (0, 0)),
            pl.BlockSpec((1, 128), lambda i: (0, 0)),
            pl.BlockSpec((128, 64), lambda i: (0, 0)),
            pl.BlockSpec((1, 64), lambda i: (0, 0)),
            pl.BlockSpec((64, 128), lambda i: (0, 0)),
            pl.BlockSpec((1, 128), lambda i: (0, 0)),
        ],
        out_specs=pl.BlockSpec((G, 128), lambda i: (0, 0)),
        out_shape=jax.ShapeDtypeStruct((G, 128), jnp.float32),
        scratch_shapes=[
            pltpu.VMEM((G, HC), jnp.float32),
            pltpu.VMEM((128, HC), jnp.float32),
            pltpu.VMEM((128, 1), jnp.float32),
        ],
    )(h_full, score2d, keep2d, batch_p, W1_p, b1, W2, b2, W3p, b3p)


# speedup vs baseline: 8.5581x; 1.0006x over previous
import jax
import jax.numpy as jnp
from jax import lax
from jax.experimental import pallas as pl
from jax.experimental.pallas import tpu as pltpu
from jax.experimental.pallas import tpu_sc as plsc

N = 10000
NP = 10240
E = 160000
D_IN = 128
D_EDGE = 16
H = 8
C = 64
HC = H * C
G = 16
RATIO = 0.8
OUT = 10
NEG = 0.2

CHUNK = 512
NCHUNK = 21
EP = 16 * NCHUNK * CHUNK
EFR = 160768
XR0 = 10240
EF0 = 2 * 10240
TR = EF0 + EFR
RANGE = 640
NROUND = NP // (2 * RANGE)
TROWS = RANGE // 16
FB = 8
B = 16
SELCAP = NCHUNK * CHUNK + 4 * B


def _perm_cols(W):
    return W.reshape(-1, H, C).transpose(0, 2, 1).reshape(-1, HC)


def _perm_vec(v):
    return v.reshape(H, C).T.reshape(HC)


def _perm_rows(M):
    return M.reshape(H, C, -1).transpose(1, 0, 2).reshape(HC, -1)


def _bcast_swap8(v):
    idx = jax.lax.iota(jnp.int32, 16) ^ 8
    dnums = lax.GatherDimensionNumbers(
        offset_dims=(), collapsed_slice_dims=(0,), start_index_map=(0,))
    return lax.gather(v, idx[:, None], dnums, (1,),
                      mode=lax.GatherScatterMode.PROMISE_IN_BOUNDS)


def _edge_body(tab, src3, attA, biasA, h_out,
               idx_c, idx_cb, sel_src, sel_dstl, sel_efi,
               idx48, dstl16, idx48b,
               rows48, rows48b,
               numb0, numb1, numb2, numb3, numb4,
               attv, biasv,
               sh0, sh1, sh2, sh3, sh4,
               sem0, sem1, sem2, sem0b, sem1b):
    c = lax.axis_index("c")
    s = lax.axis_index("s")
    numb = (numb0, numb1, numb2, numb3, numb4)
    idx_cs = (idx_c, idx_cb)
    gsemA = (sem1, sem1b)
    sh = (sh0, sh1, sh2, sh3, sh4)
    pltpu.sync_copy(attA, attv)
    pltpu.sync_copy(biasA, biasv)
    zf16 = jnp.zeros((16,), jnp.float32)
    zi16 = jnp.zeros((16,), jnp.int32)

    def _round(r, _0):
        lo = (2 * r + c) * RANGE

        def _zrow(i, _):
            for g in range(5):
                for j in range(8):
                    numb[g][i, pl.ds(j * 16, 16)] = zf16
            return 0
        lax.fori_loop(0, B, _zrow, 0)

        def _zcopy(f, _):
            cps = [pltpu.async_copy(
                numb[g].at[pl.ds(0, FB)],
                sh[g].at[pl.ds(s * TROWS + f * FB, FB)], sem1)
                for g in range(5)]
            for cp in cps:
                cp.wait()
            return 0
        lax.fori_loop(0, TROWS // FB, _zcopy, 0)
        plsc.subcore_barrier()

        def _start_chunk(ch, par):
            base = (s * NCHUNK + ch) * (3 * CHUNK)
            pltpu.async_copy(src3.at[pl.ds(base, 3 * CHUNK)],
                             idx_cs[par], gsemA[par])

        def _wait_chunk(ch, par):
            base = (s * NCHUNK + ch) * (3 * CHUNK)
            pltpu.make_async_copy(src3.at[pl.ds(base, 3 * CHUNK)],
                                  idx_cs[par], gsemA[par]).wait()

        for par in (0, 1):
            _start_chunk(jnp.int32(par), par)

        def _chunkpair(pch, cnt):
            for par in (0, 1):
                ch = 2 * pch + par
                _wait_chunk(ch, par)
                idc = idx_cs[par]
                ok = ch < NCHUNK
                for v in range(CHUNK // 16):
                    d = idc[pl.ds(CHUNK + v * 16, 16)]
                    m = (d >= lo) & (d < lo + RANGE) & ok
                    plsc.store_compressed(sel_src.at[pl.ds(cnt, 16)],
                                          idc[pl.ds(v * 16, 16)], mask=m)
                    plsc.store_compressed(sel_dstl.at[pl.ds(cnt, 16)],
                                          d - lo, mask=m)
                    plsc.store_compressed(
                        sel_efi.at[pl.ds(cnt, 16)],
                        idc[pl.ds(2 * CHUNK + v * 16, 16)], mask=m)
                    cnt = cnt + jnp.sum(m.astype(jnp.int32))
                _start_chunk(jnp.minimum(ch + 2, NCHUNK + 1), par)
            return cnt
        cnt = lax.fori_loop(0, (NCHUNK + 2) // 2, _chunkpair, jnp.int32(0))
        for par in (0, 1):
            _wait_chunk(jnp.int32(NCHUNK + 1), par)
        for t in (0, 16, 32, 48):
            sel_src[pl.ds(cnt + t, 16)] = zi16
            sel_dstl[pl.ds(cnt + t, 16)] = zi16
            sel_efi[pl.ds(cnt + t, 16)] = zi16

        nsub = (cnt + B - 1) // B
        npair = (nsub + 1) // 2
        idx48s = (idx48, idx48b)
        rows48s = (rows48, rows48b)
        gsem = (sem0, sem0b)

        def _build_idx(slot, par):
            b0 = slot * B
            idx48s[par][pl.ds(0, 16)] = sel_src[pl.ds(b0, 16)]
            idx48s[par][pl.ds(16, 16)] = (sel_dstl[pl.ds(b0, 16)]
                                          + (lo + XR0))
            idx48s[par][pl.ds(32, 16)] = sel_efi[pl.ds(b0, 16)] + EF0

        for par in (0, 1):
            _build_idx(jnp.int32(par), par)
            pltpu.async_copy(tab.at[idx48s[par]], rows48s[par], gsem[par])

        def _pair(p, _):
            for par in (0, 1):
                slot = 2 * p + par
                b0 = slot * B
                pltpu.make_async_copy(tab.at[idx48s[par]], rows48s[par],
                                      gsem[par]).wait()
                rws = rows48s[par]
                nmb = numb

                def _edge(i, _):
                    acc = zf16
                    for j in range(HC // 16):
                        jsl = pl.ds(j * 16, 16)
                        z = (rws[i, jsl] + rws[i + 16, jsl]
                             + rws[i + 32, jsl])
                        z = jnp.where(z >= 0.0, z, z * NEG)
                        acc = acc + z * attv[jsl]
                    hs = acc + _bcast_swap8(acc)
                    valid = ((b0 + i) < cnt).astype(jnp.float32)
                    ex = jnp.exp(hs) * valid
                    nmb[4][i, pl.ds(0, 16)] = ex
                    for j in range(HC // 16):
                        jsl = pl.ds(j * 16, 16)
                        nmb[j // 8][i, pl.ds((j % 8) * 16, 16)] = (
                            rws[i, jsl] * ex)
                    return 0
                lax.fori_loop(0, B, _edge, 0)
                dstl16[pl.ds(0, 16)] = sel_dstl[pl.ds(b0, 16)]
                cps = [pltpu.async_copy(nmb[g], sh[g].at[dstl16], sem2,
                                        add=True) for g in range(5)]
                for cp in cps:
                    cp.wait()
                _build_idx(slot + 2, par)
                pltpu.async_copy(tab.at[idx48s[par]], rows48s[par],
                                 gsem[par])
            return 0
        lax.fori_loop(0, npair, _pair, 0)
        for par in (0, 1):
            pltpu.make_async_copy(tab.at[idx48s[par]], rows48s[par],
                                  gsem[par]).wait()
        plsc.subcore_barrier()

        def _flush(f, _):
            r0 = s * TROWS + f * FB
            cps = [pltpu.async_copy(sh[g].at[pl.ds(r0, FB)],
                                    numb[g].at[pl.ds(0, FB)], sem1)
                   for g in range(5)]
            for cp in cps:
                cp.wait()

            def _row(row, _):
                rec = 1.0 / (numb4[row, pl.ds(0, 16)] + 1e-30)
                for j in range(HC // 16):
                    jsl = pl.ds(j * 16, 16)
                    rows48[row, jsl] = jnp.maximum(
                        numb[j // 8][row, pl.ds((j % 8) * 16, 16)] * rec
                        + biasv[jsl], 0.0)
                return 0
            lax.fori_loop(0, FB, _row, 0)
            pltpu.sync_copy(rows48.at[pl.ds(0, FB)],
                            h_out.at[pl.ds(lo + r0, FB)])
            return 0
        lax.fori_loop(0, TROWS // FB, _flush, 0)
        plsc.subcore_barrier()
        return 0

    lax.fori_loop(0, NROUND, _round, 0)


def _gat_conv_sc(tab, src3, att_p, bias_p):
    mesh = plsc.VectorSubcoreMesh(core_axis_name="c", subcore_axis_name="s")
    f32 = jnp.float32
    i32 = jnp.int32
    return pl.kernel(
        _edge_body,
        out_type=jax.ShapeDtypeStruct((NP, HC), f32),
        mesh=mesh,
        compiler_params=pltpu.CompilerParams(needs_layout_passes=False),
        scratch_types=[
            pltpu.VMEM((3 * CHUNK,), i32),
            pltpu.VMEM((3 * CHUNK,), i32),
            pltpu.VMEM((SELCAP,), i32),
            pltpu.VMEM((SELCAP,), i32),
            pltpu.VMEM((SELCAP,), i32),
            pltpu.VMEM((3 * B,), i32),
            pltpu.VMEM((16,), i32),
            pltpu.VMEM((3 * B,), i32),
            pltpu.VMEM((3 * B, HC), f32),
            pltpu.VMEM((3 * B, HC), f32),
            pltpu.VMEM((B, 128), f32),
            pltpu.VMEM((B, 128), f32),
            pltpu.VMEM((B, 128), f32),
            pltpu.VMEM((B, 128), f32),
            pltpu.VMEM((B, 128), f32),
            pltpu.VMEM((HC,), f32),
            pltpu.VMEM((HC,), f32),
            pltpu.VMEM_SHARED((RANGE, 128), f32),
            pltpu.VMEM_SHARED((RANGE, 128), f32),
            pltpu.VMEM_SHARED((RANGE, 128), f32),
            pltpu.VMEM_SHARED((RANGE, 128), f32),
            pltpu.VMEM_SHARED((RANGE, 128), f32),
            pltpu.SemaphoreType.DMA,
            pltpu.SemaphoreType.DMA,
            pltpu.SemaphoreType.DMA,
            pltpu.SemaphoreType.DMA,
            pltpu.SemaphoreType.DMA,
        ],
    )(tab, src3, att_p, bias_p)


def _fused_proj_kern(x_ref, ea_ref, wl_ref, wr_ref, we_ref, bl_ref, br_ref,
                     o_ref):
    i = pl.program_id(0)

    @pl.when(i < 10)
    def _xl():
        o_ref[...] = jnp.dot(x_ref[...], wl_ref[...],
                             preferred_element_type=jnp.float32) + bl_ref[...]

    @pl.when((i >= 10) & (i < 20))
    def _xr():
        o_ref[...] = jnp.dot(x_ref[...], wr_ref[...],
                             preferred_element_type=jnp.float32) + br_ref[...]

    @pl.when(i >= 20)
    def _ef():
        o_ref[...] = jnp.dot(ea_ref[...], we_ref[...],
                             preferred_element_type=jnp.float32)


def _fused_proj(x_pad, ea_pad, W_lp, W_rp, W_ep, b_lp, b_rp):
    nb = TR // 1024
    return pl.pallas_call(
        _fused_proj_kern,
        grid=(nb,),
        in_specs=[
            pl.BlockSpec((1024, D_IN),
                         lambda i: (jnp.where(i < 10, i,
                                              jnp.where(i < 20, i - 10, 0)),
                                    0)),
            pl.BlockSpec((1024, D_EDGE),
                         lambda i: (jnp.where(i >= 20, i - 20, 0), 0)),
            pl.BlockSpec((D_IN, HC), lambda i: (0, 0)),
            pl.BlockSpec((D_IN, HC), lambda i: (0, 0)),
            pl.BlockSpec((D_EDGE, HC), lambda i: (0, 0)),
            pl.BlockSpec((1, HC), lambda i: (0, 0)),
            pl.BlockSpec((1, HC), lambda i: (0, 0)),
        ],
        out_specs=pl.BlockSpec((1024, HC), lambda i: (i, 0)),
        out_shape=jax.ShapeDtypeStruct((TR, HC), jnp.float32),
    )(x_pad, ea_pad, W_lp, W_rp, W_ep, b_lp[None, :], b_rp[None, :])


def kernel(x, edge_index, edge_attr, batch, W_l, b_l, W_r, b_r, W_e, att,
           conv_bias, pool_p, W1, b1, W2, b2, W3, b3):
    n = N
    loop = jnp.arange(n, dtype=edge_index.dtype)
    pad = EP - (E + n)
    src_all = jnp.concatenate(
        [edge_index[0], loop, jnp.zeros((pad,), jnp.int32)])
    dst_s = jnp.concatenate(
        [edge_index[1], loop, jnp.full((pad,), 1 << 30, jnp.int32)])
    efi = jnp.concatenate(
        [jnp.arange(E, dtype=jnp.int32), jnp.full((n,), E, jnp.int32),
         jnp.zeros((pad,), jnp.int32)])
    src3 = (jnp.stack([src_all, dst_s, efi], axis=0)
            .reshape(3, EP // CHUNK, CHUNK).transpose(1, 0, 2).reshape(-1))
    src3 = jnp.concatenate([src3, jnp.zeros((2 * 3 * CHUNK,), jnp.int32)])

    W_lp, b_lp = _perm_cols(W_l), _perm_vec(b_l)
    W_rp, b_rp = _perm_cols(W_r), _perm_vec(b_r)
    W_ep = _perm_cols(W_e)
    att_p = att.T.reshape(HC)
    bias_p = _perm_vec(conv_bias)
    pool_pp = _perm_vec(pool_p)

    x_pad = jnp.pad(x, ((0, NP - N), (0, 0)))
    ea_mean = jnp.mean(edge_attr, axis=0, keepdims=True)
    ea_pad = jnp.concatenate(
        [edge_attr, jnp.tile(ea_mean, (EFR - E, 1))], axis=0)
    tab = _fused_proj(x_pad, ea_pad, W_lp, W_rp, W_ep, b_lp, b_rp)

    h_full = _gat_conv_sc(tab, src3, att_p, bias_p)

    score2d = _score_tc(h_full, pool_pp)
    batch_p = jnp.concatenate(
        [batch, jnp.full((NP - N,), 1 << 20, jnp.int32)]).reshape(NP, 1)
    keep2d = _rank_tc(score2d, batch_p)
    W1_p = jnp.concatenate([_perm_rows(W1[:HC]), _perm_rows(W1[HC:])], axis=0)
    W3p = jnp.pad(W3, ((0, 0), (0, 128 - OUT)))
    b3p = jnp.pad(b3, (0, 128 - OUT), constant_values=-1e30)
    out = _pool_mlp_tc(h_full, score2d, keep2d, batch_p,
                       W1_p, b1.reshape(1, -1), W2, b2.reshape(1, -1),
                       W3p, b3p.reshape(1, -1))
    return out[:, :OUT]


BI = 256
BJ = 2048
NI = NP // BI
NJ = NP // BJ


def _score_kern(h_ref, p_ref, o_ref):
    p = p_ref[...]
    norm = jnp.sqrt(jnp.sum(p * p))
    s = jnp.sum(h_ref[...] * p, axis=1, keepdims=True)
    o_ref[...] = jnp.tanh(s / norm)


def _score_tc(h_full, pool_pp):
    return pl.pallas_call(
        _score_kern,
        grid=(NI,),
        in_specs=[
            pl.BlockSpec((BI, HC), lambda i: (i, 0)),
            pl.BlockSpec((1, HC), lambda i: (0, 0)),
        ],
        out_specs=pl.BlockSpec((BI, 1), lambda i: (i, 0)),
        out_shape=jax.ShapeDtypeStruct((NP, 1), jnp.float32),
    )(h_full, pool_pp.reshape(1, HC))


def _rank_kern(si_ref, sj_ref, bi_ref, bj_ref, keep_ref, rank_acc, cnt_acc):
    i = pl.program_id(0)
    j = pl.program_id(1)

    @pl.when(j == 0)
    def _init():
        rank_acc[...] = jnp.zeros_like(rank_acc)
        cnt_acc[...] = jnp.zeros_like(cnt_acc)

    si = si_ref[...]
    sj = sj_ref[...]
    bi = bi_ref[...]
    bj = bj_ref[...]
    ii = (jax.lax.broadcasted_iota(jnp.int32, (BI, 1), 0)
          + i * BI).astype(jnp.float32)
    jj = (jax.lax.broadcasted_iota(jnp.int32, (1, BJ), 1)
          + j * BJ).astype(jnp.float32)
    same = (bi == bj)
    beats = same & ((sj > si) | ((sj == si) & (jj < ii)))
    rank_acc[...] += jnp.sum(beats.astype(jnp.float32), axis=1,
                             keepdims=True)
    cnt_acc[...] += jnp.sum(same.astype(jnp.float32), axis=1, keepdims=True)

    @pl.when(j == NJ - 1)
    def _fin():
        k = jnp.floor((4.0 * cnt_acc[...] + 4.0) * 0.2)
        keep_ref[...] = (rank_acc[...] < k).astype(jnp.float32)


def _rank_tc(score2d, batch_p):
    return pl.pallas_call(
        _rank_kern,
        grid=(NI, NJ),
        in_specs=[
            pl.BlockSpec((BI, 1), lambda i, j: (i, 0)),
            pl.BlockSpec((1, BJ), lambda i, j: (0, j)),
            pl.BlockSpec((BI, 1), lambda i, j: (i, 0)),
            pl.BlockSpec((1, BJ), lambda i, j: (0, j)),
        ],
        out_specs=pl.BlockSpec((BI, 1), lambda i, j: (i, 0)),
        out_shape=jax.ShapeDtypeStruct((NP, 1), jnp.float32),
        scratch_shapes=[
            pltpu.VMEM((BI, 1), jnp.float32),
            pltpu.VMEM((BI, 1), jnp.float32),
        ],
    )(score2d, score2d.reshape(1, NP), batch_p, batch_p.reshape(1, NP))


def _pool_kern(h_ref, s_ref, k_ref, b_ref, w1_ref, b1_ref, w2_ref, b2_ref,
               w3_ref, b3_ref, o_ref, gmp_acc, gap_acc, cnt_acc):
    i = pl.program_id(0)

    @pl.when(i == 0)
    def _init():
        gmp_acc[...] = jnp.full_like(gmp_acc, -jnp.inf)
        gap_acc[...] = jnp.zeros_like(gap_acc)
        cnt_acc[...] = jnp.zeros_like(cnt_acc)

    h = h_ref[...]
    sc = s_ref[...]
    kp = k_ref[...]
    bt = b_ref[...]
    val = h * sc
    g_iota = jax.lax.broadcasted_iota(jnp.int32, (1, 128), 1)
    onehot = ((bt == g_iota) & (kp > 0.0)).astype(jnp.float32)
    valk = val * kp
    gap_acc[...] += jax.lax.dot_general(
        onehot, valk, (((0,), (0,)), ((), ())),
        preferred_element_type=jnp.float32)
    cnt_acc[...] += jax.lax.dot_general(
        onehot, kp, (((0,), (0,)), ((), ())),
        preferred_element_type=jnp.float32)
    for g in range(G):
        mg = (bt == g) & (kp > 0.0)
        masked = jnp.where(mg, val, jnp.float32(-1e30))
        gmp_acc[pl.ds(g, 1), :] = jnp.maximum(
            gmp_acc[pl.ds(g, 1), :],
            jnp.max(masked, axis=0, keepdims=True))

    @pl.when(i == NI - 1)
    def _fin():
        gmp = gmp_acc[...]
        gap = gap_acc[pl.ds(0, G), :] / (cnt_acc[pl.ds(0, G), :] + 1e-16)
        cont = jnp.concatenate([gmp, gap], axis=1)
        z = jnp.maximum(jnp.dot(cont, w1_ref[...],
                                preferred_element_type=jnp.float32)
                        + b1_ref[...], 0.0)
        z = jnp.maximum(jnp.dot(z, w2_ref[...],
                                preferred_element_type=jnp.float32)
                        + b2_ref[...], 0.0)
        z = jnp.dot(z, w3_ref[...],
                    preferred_element_type=jnp.float32) + b3_ref[...]
        zmax = jnp.max(z, axis=1, keepdims=True)
        zs = z - zmax
        o_ref[...] = zs - jnp.log(jnp.sum(jnp.exp(zs), axis=1,
                                          keepdims=True))


def _pool_mlp_tc(h_full, score2d, keep2d, batch_p, W1_p, b1, W2, b2, W3p,
                 b3p):
    return pl.pallas_call(
        _pool_kern,
        grid=(NI,),
        in_specs=[
            pl.BlockSpec((BI, HC), lambda i: (i, 0)),
            pl.BlockSpec((BI, 1), lambda i: (i, 0)),
            pl.BlockSpec((BI, 1), lambda i: (i, 0)),
            pl.BlockSpec((BI, 1), lambda i: (i, 0)),
            pl.BlockSpec((2 * HC, 128), lambda i: (0, 0)),
            pl.BlockSpec((1, 128), lambda i: (0, 0)),
            pl.BlockSpec((128, 64), lambda i: (0, 0)),
            pl.BlockSpec((1, 64), lambda i: (0, 0)),
            pl.BlockSpec((64, 128), lambda i: (0, 0)),
            pl.BlockSpec((1, 128), lambda i: (0, 0)),
        ],
        out_specs=pl.BlockSpec((G, 128), lambda i: (0, 0)),
        out_shape=jax.ShapeDtypeStruct((G, 128), jnp.float32),
        scratch_shapes=[
            pltpu.VMEM((G, HC), jnp.float32),
            pltpu.VMEM((128, HC), jnp.float32),
            pltpu.VMEM((128, 1), jnp.float32),
        ],
    )(h_full, score2d, keep2d, batch_p, W1_p, b1, W2, b2, W3p, b3p)
